# Initial kernel scaffold; baseline (speedup 1.0000x reference)
import jax
import jax.numpy as jnp
from jax.experimental import pallas as pl


def kernel(x, edge_index, edge_attr, params):
    raise NotImplementedError("write your pallas kernel here")


# speedup vs baseline: 2.0437x; 2.0437x over previous
import functools

import jax
import jax.numpy as jnp
from jax import lax
from jax.experimental import pallas as pl
from jax.experimental.pallas import tpu as pltpu
from jax.experimental.pallas import tpu_sc as plsc

_N = 10000
_E = 320000
_D = 128
_LN_EPS = 1e-5

_BE = 2000
_BN = 2000
_NW = 32
_GCH = 80
_SCH = 80


def _ln_silu(h, g, b):
    mu = jnp.mean(h, axis=-1, keepdims=True)
    c = h - mu
    var = jnp.mean(c * c, axis=-1, keepdims=True)
    hn = c * lax.rsqrt(var + _LN_EPS) * g + b
    return hn * jax.nn.sigmoid(hn)


def _const_spec(shape):
    return pl.BlockSpec(shape, lambda i: tuple(0 for _ in shape))


def _linear_pl(x, w, b, block):
    n, k = x.shape
    m = w.shape[1]
    def body(x_ref, w_ref, b_ref, o_ref):
        o_ref[...] = (
            jnp.dot(x_ref[...], w_ref[...], preferred_element_type=jnp.float32)
            + b_ref[...])
    return pl.pallas_call(
        body,
        grid=(n // block,),
        in_specs=[pl.BlockSpec((block, k), lambda i: (i, 0)),
                  _const_spec((k, m)),
                  _const_spec((1, m))],
        out_specs=pl.BlockSpec((block, m), lambda i: (i, 0)),
        out_shape=jax.ShapeDtypeStruct((n, m), jnp.float32),
    )(x, w, b.reshape(1, m))


def _message_pl(g, e, wd, ws, we, b1, g1, t1, w2m, b2m, w2g, b2g):
    nb = _E // _BE

    def body(hd_ref, hs_ref, e_ref, wd_, ws_, we_, b1_, g1_, t1_,
             w2m_, b2m_, w2g_, b2g_, o_ref):
        h1 = (jnp.dot(hd_ref[...], wd_[...], preferred_element_type=jnp.float32)
              + jnp.dot(hs_ref[...], ws_[...], preferred_element_type=jnp.float32)
              + jnp.dot(e_ref[...], we_[...], preferred_element_type=jnp.float32)
              + b1_[...])
        hm = _ln_silu(h1[:, :_D], g1_[:, :_D], t1_[:, :_D])
        hg = _ln_silu(h1[:, _D:], g1_[:, _D:], t1_[:, _D:])
        msg = jnp.dot(hm, w2m_[...], preferred_element_type=jnp.float32) + b2m_[...]
        gl = jnp.dot(hg, w2g_[...], preferred_element_type=jnp.float32) + b2g_[...]
        o_ref[...] = jax.nn.sigmoid(gl) * msg

    return pl.pallas_call(
        body,
        grid=(nb,),
        in_specs=[pl.BlockSpec((_BE, _D), lambda i: (i + nb, 0)),
                  pl.BlockSpec((_BE, _D), lambda i: (i, 0)),
                  pl.BlockSpec((_BE, _D), lambda i: (i, 0)),
                  _const_spec((_D, 2 * _D)), _const_spec((_D, 2 * _D)),
                  _const_spec((_D, 2 * _D)), _const_spec((1, 2 * _D)),
                  _const_spec((1, 2 * _D)), _const_spec((1, 2 * _D)),
                  _const_spec((_D, _D)), _const_spec((1, _D)),
                  _const_spec((_D, _D)), _const_spec((1, _D))],
        out_specs=pl.BlockSpec((_BE, _D), lambda i: (i, 0)),
        out_shape=jax.ShapeDtypeStruct((_E, _D), jnp.float32),
    )(g, g, e, wd, ws, we, b1, g1, t1, w2m, b2m, w2g, b2g)


def _node_pl(h, parts, wh, wa, b1, g1, t1, w2, b2):
    nb = _N // _BN

    def body(h_ref, p0_ref, p1_ref, wh_, wa_, b1_, g1_, t1_, w2_, b2_, o_ref):
        aggr = p0_ref[...] + p1_ref[...]
        h1 = (jnp.dot(h_ref[...], wh_[...], preferred_element_type=jnp.float32)
              + jnp.dot(aggr, wa_[...], preferred_element_type=jnp.float32)
              + b1_[...])
        h1 = _ln_silu(h1, g1_[...], t1_[...])
        o_ref[...] = (jnp.dot(h1, w2_[...], preferred_element_type=jnp.float32)
                      + b2_[...] + h_ref[...])

    return pl.pallas_call(
        body,
        grid=(nb,),
        in_specs=[pl.BlockSpec((_BN, _D), lambda i: (i, 0)),
                  pl.BlockSpec((_BN, _D), lambda i: (i, 0)),
                  pl.BlockSpec((_BN, _D), lambda i: (i + nb, 0)),
                  _const_spec((_D, _D)), _const_spec((_D, _D)),
                  _const_spec((1, _D)), _const_spec((1, _D)),
                  _const_spec((1, _D)), _const_spec((_D, _D)),
                  _const_spec((1, _D))],
        out_specs=pl.BlockSpec((_BN, _D), lambda i: (i, 0)),
        out_shape=jax.ShapeDtypeStruct((_N, _D), jnp.float32),
    )(h, parts, parts, wh, wa, b1, g1, t1, w2, b2)


def _edgeupd_pl(e, g, we, ws, wd, b1, g1, t1, w2, b2):
    nb = _E // _BE

    def body(e_ref, ns_ref, nd_ref, we_, ws_, wd_, b1_, g1_, t1_, w2_, b2_, o_ref):
        h1 = (jnp.dot(e_ref[...], we_[...], preferred_element_type=jnp.float32)
              + jnp.dot(ns_ref[...], ws_[...], preferred_element_type=jnp.float32)
              + jnp.dot(nd_ref[...], wd_[...], preferred_element_type=jnp.float32)
              + b1_[...])
        h1 = _ln_silu(h1, g1_[...], t1_[...])
        o_ref[...] = (jnp.dot(h1, w2_[...], preferred_element_type=jnp.float32)
                      + b2_[...] + e_ref[...])

    return pl.pallas_call(
        body,
        grid=(nb,),
        in_specs=[pl.BlockSpec((_BE, _D), lambda i: (i, 0)),
                  pl.BlockSpec((_BE, _D), lambda i: (i, 0)),
                  pl.BlockSpec((_BE, _D), lambda i: (i + nb, 0)),
                  _const_spec((_D, _D)), _const_spec((_D, _D)),
                  _const_spec((_D, _D)), _const_spec((1, _D)),
                  _const_spec((1, _D)), _const_spec((1, _D)),
                  _const_spec((_D, _D)), _const_spec((1, _D))],
        out_specs=pl.BlockSpec((_BE, _D), lambda i: (i, 0)),
        out_shape=jax.ShapeDtypeStruct((_E, _D), jnp.float32),
    )(e, g, g, we, ws, wd, b1, g1, t1, w2, b2)


def _sc_gather(h, idx2):
    b = idx2.shape[0]
    per_w = b // _NW
    nch = per_w // _GCH
    mesh = plsc.VectorSubcoreMesh(core_axis_name="c", subcore_axis_name="s")

    @functools.partial(
        pl.kernel, mesh=mesh,
        out_type=jax.ShapeDtypeStruct((b, _D), jnp.float32),
        scratch_types=[pltpu.VMEM((_GCH,), jnp.int32),
                       pltpu.VMEM((_GCH, _D), jnp.float32),
                       pltpu.SemaphoreType.DMA],
    )
    def k(h_hbm, idx_hbm, out_hbm, idx_v, rows_v, sem):
        wid = lax.axis_index("s") * 2 + lax.axis_index("c")
        base = wid * per_w

        def body(i, carry):
            off = base + i * _GCH
            pltpu.sync_copy(idx_hbm.at[pl.ds(off, _GCH)], idx_v)
            pltpu.async_copy(h_hbm.at[idx_v], rows_v, sem).wait()
            pltpu.sync_copy(rows_v, out_hbm.at[pl.ds(off, _GCH)])
            return carry

        lax.fori_loop(0, nch, body, 0)

    return k(h, idx2)


def _sc_scatter(m, dst, zer):
    per_w = _E // _NW
    nch = per_w // _SCH
    stripe = 624
    tail = _N - 16 * stripe
    mesh = plsc.VectorSubcoreMesh(core_axis_name="c", subcore_axis_name="s")

    @functools.partial(
        pl.kernel, mesh=mesh,
        out_type=jax.ShapeDtypeStruct((2 * _N, _D), jnp.float32),
        scratch_types=[pltpu.VMEM((_SCH,), jnp.int32),
                       pltpu.VMEM((_SCH, _D), jnp.float32),
                       pltpu.VMEM_SHARED((_N, _D), jnp.float32)],
    )
    def k(m_hbm, dst_hbm, zer_hbm, out_hbm, idx_v, rows_v, acc):
        cid = lax.axis_index("c")
        sid = lax.axis_index("s")
        wid = sid * 2 + cid
        pltpu.sync_copy(zer_hbm.at[pl.ds(sid * stripe, stripe)],
                        acc.at[pl.ds(sid * stripe, stripe)])
        @pl.when(sid == 15)
        def _():
            pltpu.sync_copy(zer_hbm.at[pl.ds(16 * stripe, tail)],
                            acc.at[pl.ds(16 * stripe, tail)])
        plsc.subcore_barrier()

        base = wid * per_w

        def body(i, carry):
            off = base + i * _SCH
            pltpu.sync_copy(dst_hbm.at[pl.ds(off, _SCH)], idx_v)
            pltpu.sync_copy(m_hbm.at[pl.ds(off, _SCH)], rows_v)
            pltpu.sync_copy(rows_v, acc.at[idx_v], add=True)
            return carry

        lax.fori_loop(0, nch, body, 0)
        plsc.subcore_barrier()
        pltpu.sync_copy(
            acc.at[pl.ds(sid * stripe, stripe)],
            out_hbm.at[pl.ds(cid * _N + sid * stripe, stripe)])
        @pl.when(sid == 15)
        def _():
            pltpu.sync_copy(
                acc.at[pl.ds(16 * stripe, tail)],
                out_hbm.at[pl.ds(cid * _N + 16 * stripe, tail)])

    return k(m, dst, zer)


def _pack_layer(lp):
    mp, gp, np_, ep = lp["message"], lp["gate"], lp["node"], lp["edge"]
    w1 = jnp.concatenate([mp["l1"]["W"], gp["l1"]["W"]], axis=1)
    msg = dict(
        wd=w1[:_D], ws=w1[_D:2 * _D], we=w1[2 * _D:],
        b1=jnp.concatenate([mp["l1"]["b"], gp["l1"]["b"]]).reshape(1, 2 * _D),
        g1=jnp.concatenate([mp["ln_g"], gp["ln_g"]]).reshape(1, 2 * _D),
        t1=jnp.concatenate([mp["ln_b"], gp["ln_b"]]).reshape(1, 2 * _D),
        w2m=mp["l2"]["W"], b2m=mp["l2"]["b"].reshape(1, _D),
        w2g=jnp.broadcast_to(gp["l2"]["W"], (_D, _D)),
        b2g=jnp.broadcast_to(gp["l2"]["b"].reshape(1, 1), (1, _D)),
    )
    node = dict(
        wh=np_["l1"]["W"][:_D], wa=np_["l1"]["W"][_D:],
        b1=np_["l1"]["b"].reshape(1, _D),
        g1=np_["ln_g"].reshape(1, _D), t1=np_["ln_b"].reshape(1, _D),
        w2=np_["l2"]["W"], b2=np_["l2"]["b"].reshape(1, _D),
    )
    edge = dict(
        we=ep["l1"]["W"][:_D], ws=ep["l1"]["W"][_D:2 * _D],
        wd=ep["l1"]["W"][2 * _D:],
        b1=ep["l1"]["b"].reshape(1, _D),
        g1=ep["ln_g"].reshape(1, _D), t1=ep["ln_b"].reshape(1, _D),
        w2=ep["l2"]["W"], b2=ep["l2"]["b"].reshape(1, _D),
    )
    return msg, node, edge


def kernel(x, edge_index, edge_attr, params):
    src = edge_index[0]
    dst = edge_index[1]
    idx2 = jnp.concatenate([src, dst])
    zer = jnp.zeros((_N, _D), jnp.float32)

    h = _linear_pl(x, params["node_enc"]["W"], params["node_enc"]["b"], _BN)
    e = _linear_pl(edge_attr, params["edge_enc"]["W"], params["edge_enc"]["b"], _BE)

    g = _sc_gather(h, idx2)
    for lp in params["layers"]:
        msg, node, edge = _pack_layer(lp)
        m = _message_pl(g, e, **msg)
        parts = _sc_scatter(m, dst, zer)
        h = _node_pl(h, parts, **node)
        g = _sc_gather(h, idx2)
        e = _edgeupd_pl(e, g, **edge)

    x_out = _linear_pl(h, params["node_dec"]["W"], params["node_dec"]["b"], _BN)
    e_out = _linear_pl(e, params["edge_dec"]["W"], params["edge_dec"]["b"], _BE)
    return (x_out, e_out)


# speedup vs baseline: 2.6358x; 1.2897x over previous
import functools

import jax
import jax.numpy as jnp
from jax import lax
from jax.experimental import pallas as pl
from jax.experimental.pallas import tpu as pltpu
from jax.experimental.pallas import tpu_sc as plsc

_N = 10000
_E = 320000
_D = 128
_LN_EPS = 1e-5

_BE = 2000
_BN = 2000
_NW = 32
_GCH = 80
_SCH = 80


def _ln_silu(h, g, b):
    mu = jnp.mean(h, axis=-1, keepdims=True)
    c = h - mu
    var = jnp.mean(c * c, axis=-1, keepdims=True)
    hn = c * lax.rsqrt(var + _LN_EPS) * g + b
    return hn * jax.nn.sigmoid(hn)


def _const_spec(shape):
    return pl.BlockSpec(shape, lambda i: tuple(0 for _ in shape))


def _linear_pl(x, w, b, block):
    n, k = x.shape
    m = w.shape[1]
    def body(x_ref, w_ref, b_ref, o_ref):
        o_ref[...] = (
            jnp.dot(x_ref[...], w_ref[...], preferred_element_type=jnp.float32)
            + b_ref[...])
    return pl.pallas_call(
        body,
        grid=(n // block,),
        in_specs=[pl.BlockSpec((block, k), lambda i: (i, 0)),
                  _const_spec((k, m)),
                  _const_spec((1, m))],
        out_specs=pl.BlockSpec((block, m), lambda i: (i, 0)),
        out_shape=jax.ShapeDtypeStruct((n, m), jnp.float32),
    )(x, w, b.reshape(1, m))


def _message_pl(g, e, wd, ws, we, b1, g1, t1, w2m, b2m, w2g, b2g):
    nb = _E // _BE

    def body(hd_ref, hs_ref, e_ref, wd_, ws_, we_, b1_, g1_, t1_,
             w2m_, b2m_, w2g_, b2g_, o_ref):
        h1 = (jnp.dot(hd_ref[...], wd_[...], preferred_element_type=jnp.float32)
              + jnp.dot(hs_ref[...], ws_[...], preferred_element_type=jnp.float32)
              + jnp.dot(e_ref[...], we_[...], preferred_element_type=jnp.float32)
              + b1_[...])
        hm = _ln_silu(h1[:, :_D], g1_[:, :_D], t1_[:, :_D])
        hg = _ln_silu(h1[:, _D:], g1_[:, _D:], t1_[:, _D:])
        msg = jnp.dot(hm, w2m_[...], preferred_element_type=jnp.float32) + b2m_[...]
        gl = jnp.dot(hg, w2g_[...], preferred_element_type=jnp.float32) + b2g_[...]
        o_ref[...] = jax.nn.sigmoid(gl) * msg

    return pl.pallas_call(
        body,
        grid=(nb,),
        in_specs=[pl.BlockSpec((_BE, _D), lambda i: (i + nb, 0)),
                  pl.BlockSpec((_BE, _D), lambda i: (i, 0)),
                  pl.BlockSpec((_BE, _D), lambda i: (i, 0)),
                  _const_spec((_D, 2 * _D)), _const_spec((_D, 2 * _D)),
                  _const_spec((_D, 2 * _D)), _const_spec((1, 2 * _D)),
                  _const_spec((1, 2 * _D)), _const_spec((1, 2 * _D)),
                  _const_spec((_D, _D)), _const_spec((1, _D)),
                  _const_spec((_D, _D)), _const_spec((1, _D))],
        out_specs=pl.BlockSpec((_BE, _D), lambda i: (i, 0)),
        out_shape=jax.ShapeDtypeStruct((_E, _D), jnp.float32),
    )(g, g, e, wd, ws, we, b1, g1, t1, w2m, b2m, w2g, b2g)


def _node_pl(h, parts, wh, wa, b1, g1, t1, w2, b2):
    nb = _N // _BN

    def body(h_ref, p0_ref, p1_ref, wh_, wa_, b1_, g1_, t1_, w2_, b2_, o_ref):
        aggr = p0_ref[...] + p1_ref[...]
        h1 = (jnp.dot(h_ref[...], wh_[...], preferred_element_type=jnp.float32)
              + jnp.dot(aggr, wa_[...], preferred_element_type=jnp.float32)
              + b1_[...])
        h1 = _ln_silu(h1, g1_[...], t1_[...])
        o_ref[...] = (jnp.dot(h1, w2_[...], preferred_element_type=jnp.float32)
                      + b2_[...] + h_ref[...])

    return pl.pallas_call(
        body,
        grid=(nb,),
        in_specs=[pl.BlockSpec((_BN, _D), lambda i: (i, 0)),
                  pl.BlockSpec((_BN, _D), lambda i: (i, 0)),
                  pl.BlockSpec((_BN, _D), lambda i: (i + nb, 0)),
                  _const_spec((_D, _D)), _const_spec((_D, _D)),
                  _const_spec((1, _D)), _const_spec((1, _D)),
                  _const_spec((1, _D)), _const_spec((_D, _D)),
                  _const_spec((1, _D))],
        out_specs=pl.BlockSpec((_BN, _D), lambda i: (i, 0)),
        out_shape=jax.ShapeDtypeStruct((_N, _D), jnp.float32),
    )(h, parts, parts, wh, wa, b1, g1, t1, w2, b2)


def _edgeupd_pl(e, g, we, ws, wd, b1, g1, t1, w2, b2):
    nb = _E // _BE

    def body(e_ref, ns_ref, nd_ref, we_, ws_, wd_, b1_, g1_, t1_, w2_, b2_, o_ref):
        h1 = (jnp.dot(e_ref[...], we_[...], preferred_element_type=jnp.float32)
              + jnp.dot(ns_ref[...], ws_[...], preferred_element_type=jnp.float32)
              + jnp.dot(nd_ref[...], wd_[...], preferred_element_type=jnp.float32)
              + b1_[...])
        h1 = _ln_silu(h1, g1_[...], t1_[...])
        o_ref[...] = (jnp.dot(h1, w2_[...], preferred_element_type=jnp.float32)
                      + b2_[...] + e_ref[...])

    return pl.pallas_call(
        body,
        grid=(nb,),
        in_specs=[pl.BlockSpec((_BE, _D), lambda i: (i, 0)),
                  pl.BlockSpec((_BE, _D), lambda i: (i, 0)),
                  pl.BlockSpec((_BE, _D), lambda i: (i + nb, 0)),
                  _const_spec((_D, _D)), _const_spec((_D, _D)),
                  _const_spec((_D, _D)), _const_spec((1, _D)),
                  _const_spec((1, _D)), _const_spec((1, _D)),
                  _const_spec((_D, _D)), _const_spec((1, _D))],
        out_specs=pl.BlockSpec((_BE, _D), lambda i: (i, 0)),
        out_shape=jax.ShapeDtypeStruct((_E, _D), jnp.float32),
    )(e, g, g, we, ws, wd, b1, g1, t1, w2, b2)


def _sc_gather(h, idx2):
    b = idx2.shape[0]
    per_w = b // _NW
    nch = per_w // _GCH
    mesh = plsc.VectorSubcoreMesh(core_axis_name="c", subcore_axis_name="s")

    @functools.partial(
        pl.kernel, mesh=mesh,
        out_type=jax.ShapeDtypeStruct((b, _D), jnp.float32),
        scratch_types=[pltpu.VMEM((per_w,), jnp.int32),
                       pltpu.VMEM((_GCH, _D), jnp.float32),
                       pltpu.VMEM((_GCH, _D), jnp.float32),
                       pltpu.SemaphoreType.DMA, pltpu.SemaphoreType.DMA,
                       pltpu.SemaphoreType.DMA, pltpu.SemaphoreType.DMA],
    )
    def k(h_hbm, idx_hbm, out_hbm, idx_v, r0, r1, g0, g1, w0, w1):
        wid = lax.axis_index("s") * 2 + lax.axis_index("c")
        base = wid * per_w
        pltpu.sync_copy(idx_hbm.at[pl.ds(base, per_w)], idx_v)
        rows = (r0, r1)
        gsem = (g0, g1)
        wsem = (w0, w1)

        def outer(j, carry):
            for t in range(2):
                i = j * 2 + t

                @pl.when(j > 0)
                def _():
                    pltpu.make_async_copy(
                        rows[t], out_hbm.at[pl.ds(base + i * _GCH, _GCH)],
                        wsem[t]).wait()

                pltpu.async_copy(
                    h_hbm.at[idx_v.at[pl.ds(i * _GCH, _GCH)]], rows[t],
                    gsem[t])
            for t in range(2):
                i = j * 2 + t
                pltpu.make_async_copy(
                    h_hbm.at[idx_v.at[pl.ds(i * _GCH, _GCH)]], rows[t],
                    gsem[t]).wait()
                pltpu.async_copy(rows[t],
                                 out_hbm.at[pl.ds(base + i * _GCH, _GCH)],
                                 wsem[t])
            return carry

        lax.fori_loop(0, nch // 2, outer, 0)
        for t in range(2):
            pltpu.make_async_copy(
                rows[t], out_hbm.at[pl.ds(base, _GCH)], wsem[t]).wait()

    return k(h, idx2)


def _sc_scatter(m, dst, zer):
    per_w = _E // _NW
    nch = per_w // _SCH
    stripe = 624
    tail = _N - 16 * stripe
    mesh = plsc.VectorSubcoreMesh(core_axis_name="c", subcore_axis_name="s")

    @functools.partial(
        pl.kernel, mesh=mesh,
        out_type=jax.ShapeDtypeStruct((2 * _N, _D), jnp.float32),
        scratch_types=[pltpu.VMEM((_SCH,), jnp.int32),
                       pltpu.VMEM((_SCH,), jnp.int32),
                       pltpu.VMEM((_SCH, _D), jnp.float32),
                       pltpu.VMEM((_SCH, _D), jnp.float32),
                       pltpu.VMEM_SHARED((_N, _D), jnp.float32),
                       pltpu.SemaphoreType.DMA, pltpu.SemaphoreType.DMA,
                       pltpu.SemaphoreType.DMA, pltpu.SemaphoreType.DMA],
    )
    def k(m_hbm, dst_hbm, zer_hbm, out_hbm, i0, i1, r0, r1, acc,
          mi0, mi1, mr0, mr1):
        cid = lax.axis_index("c")
        sid = lax.axis_index("s")
        wid = sid * 2 + cid
        pltpu.sync_copy(zer_hbm.at[pl.ds(sid * stripe, stripe)],
                        acc.at[pl.ds(sid * stripe, stripe)])
        @pl.when(sid == 15)
        def _():
            pltpu.sync_copy(zer_hbm.at[pl.ds(16 * stripe, tail)],
                            acc.at[pl.ds(16 * stripe, tail)])
        plsc.subcore_barrier()

        base = wid * per_w
        idx_b = (i0, i1)
        rows_b = (r0, r1)
        isem = (mi0, mi1)
        msem = (mr0, mr1)

        lead = nch % 2
        if lead:
            pltpu.sync_copy(dst_hbm.at[pl.ds(base, _SCH)], i0)
            pltpu.sync_copy(m_hbm.at[pl.ds(base, _SCH)], r0)
            pltpu.sync_copy(r0, acc.at[i0], add=True)

        def outer(j, carry):
            for t in range(2):
                i = lead + j * 2 + t
                off = base + i * _SCH
                pltpu.async_copy(dst_hbm.at[pl.ds(off, _SCH)], idx_b[t],
                                 isem[t])
                pltpu.async_copy(m_hbm.at[pl.ds(off, _SCH)], rows_b[t],
                                 msem[t])
            for t in range(2):
                i = lead + j * 2 + t
                off = base + i * _SCH
                pltpu.make_async_copy(dst_hbm.at[pl.ds(off, _SCH)], idx_b[t],
                                      isem[t]).wait()
                pltpu.make_async_copy(m_hbm.at[pl.ds(off, _SCH)], rows_b[t],
                                      msem[t]).wait()
                pltpu.sync_copy(rows_b[t], acc.at[idx_b[t]], add=True)
            return carry

        lax.fori_loop(0, (nch - lead) // 2, outer, 0)
        plsc.subcore_barrier()
        pltpu.sync_copy(
            acc.at[pl.ds(sid * stripe, stripe)],
            out_hbm.at[pl.ds(cid * _N + sid * stripe, stripe)])
        @pl.when(sid == 15)
        def _():
            pltpu.sync_copy(
                acc.at[pl.ds(16 * stripe, tail)],
                out_hbm.at[pl.ds(cid * _N + 16 * stripe, tail)])

    return k(m, dst, zer)


def _pack_layer(lp):
    mp, gp, np_, ep = lp["message"], lp["gate"], lp["node"], lp["edge"]
    w1 = jnp.concatenate([mp["l1"]["W"], gp["l1"]["W"]], axis=1)
    msg = dict(
        wd=w1[:_D], ws=w1[_D:2 * _D], we=w1[2 * _D:],
        b1=jnp.concatenate([mp["l1"]["b"], gp["l1"]["b"]]).reshape(1, 2 * _D),
        g1=jnp.concatenate([mp["ln_g"], gp["ln_g"]]).reshape(1, 2 * _D),
        t1=jnp.concatenate([mp["ln_b"], gp["ln_b"]]).reshape(1, 2 * _D),
        w2m=mp["l2"]["W"], b2m=mp["l2"]["b"].reshape(1, _D),
        w2g=jnp.broadcast_to(gp["l2"]["W"], (_D, _D)),
        b2g=jnp.broadcast_to(gp["l2"]["b"].reshape(1, 1), (1, _D)),
    )
    node = dict(
        wh=np_["l1"]["W"][:_D], wa=np_["l1"]["W"][_D:],
        b1=np_["l1"]["b"].reshape(1, _D),
        g1=np_["ln_g"].reshape(1, _D), t1=np_["ln_b"].reshape(1, _D),
        w2=np_["l2"]["W"], b2=np_["l2"]["b"].reshape(1, _D),
    )
    edge = dict(
        we=ep["l1"]["W"][:_D], ws=ep["l1"]["W"][_D:2 * _D],
        wd=ep["l1"]["W"][2 * _D:],
        b1=ep["l1"]["b"].reshape(1, _D),
        g1=ep["ln_g"].reshape(1, _D), t1=ep["ln_b"].reshape(1, _D),
        w2=ep["l2"]["W"], b2=ep["l2"]["b"].reshape(1, _D),
    )
    return msg, node, edge


def kernel(x, edge_index, edge_attr, params):
    src = edge_index[0]
    dst = edge_index[1]
    idx2 = jnp.concatenate([src, dst])
    zer = jnp.zeros((_N, _D), jnp.float32)

    h = _linear_pl(x, params["node_enc"]["W"], params["node_enc"]["b"], _BN)
    e = _linear_pl(edge_attr, params["edge_enc"]["W"], params["edge_enc"]["b"], _BE)

    g = _sc_gather(h, idx2)
    for lp in params["layers"]:
        msg, node, edge = _pack_layer(lp)
        m = _message_pl(g, e, **msg)
        parts = _sc_scatter(m, dst, zer)
        h = _node_pl(h, parts, **node)
        g = _sc_gather(h, idx2)
        e = _edgeupd_pl(e, g, **edge)

    x_out = _linear_pl(h, params["node_dec"]["W"], params["node_dec"]["b"], _BN)
    e_out = _linear_pl(e, params["edge_dec"]["W"], params["edge_dec"]["b"], _BE)
    return (x_out, e_out)
